# baseline (speedup 1.0000x reference)
import functools

import jax
import jax.numpy as jnp
from jax import lax
from jax.experimental import pallas as pl
from jax.experimental.pallas import tpu as pltpu
from jax.experimental.pallas import tpu_sc as plsc

N = 10000
E = 160000
DN = 256
DZ = 512
G = 64
EPS = 1e-5

DP = 256
NC, NS = 2, 16
NW = NC * NS
EPW = E // NW
CG = 40
EPT = E // NS
CS = 80
ZC = 80
NZCH = N // ZC

_mesh = plsc.VectorSubcoreMesh(
    core_axis_name="c", subcore_axis_name="s", num_cores=NC, num_subcores=NS)


EH_A = 81920
EH_B = E - EH_A
EPW_A = EH_A // NW
EPW_B = EH_B // NW
EPT_A = EH_A // NS
EPT_B = EH_B // NS


def _make_gather(glob_off, epw, ne):
    nch = epw // CG

    def body(td, ts, dst, src, gd, gs, idx_d, idx_s,
             rd0, rs0, rd1, rs1, g0, g1, w0, w1):
        c = lax.axis_index("c")
        s = lax.axis_index("s")
        wid = s * NC + c
        base = glob_off + wid * epw
        obase = wid * epw
        pltpu.sync_copy(dst.at[pl.ds(base, epw)], idx_d)
        pltpu.sync_copy(src.at[pl.ds(base, epw)], idx_s)

        def issue_gather(k, rd, rs, sem):
            pltpu.async_copy(td.at[idx_d.at[pl.ds(k * CG, CG)]], rd, sem)
            pltpu.async_copy(ts.at[idx_s.at[pl.ds(k * CG, CG)]], rs, sem)

        def wait_gather(rd, rs, sem):
            pltpu.make_async_copy(td.at[idx_d.at[pl.ds(0, CG)]], rd, sem).wait()
            pltpu.make_async_copy(ts.at[idx_s.at[pl.ds(0, CG)]], rs, sem).wait()

        def issue_write(k, rd, rs, sem):
            off = obase + k * CG
            pltpu.async_copy(rd, gd.at[pl.ds(off, CG)], sem)
            pltpu.async_copy(rs, gs.at[pl.ds(off, CG)], sem)

        def wait_write(rd, rs, sem):
            pltpu.make_async_copy(rd, gd.at[pl.ds(obase, CG)], sem).wait()
            pltpu.make_async_copy(rs, gs.at[pl.ds(obase, CG)], sem).wait()

        issue_gather(0, rd0, rs0, g0)

        def step(k, carry):
            def phase(rd_c, rs_c, g_c, w_c, rd_n, rs_n, g_n, w_n):
                @pl.when(k > 0)
                def _():
                    wait_write(rd_n, rs_n, w_n)

                @pl.when(k < nch - 1)
                def _():
                    issue_gather(k + 1, rd_n, rs_n, g_n)

                wait_gather(rd_c, rs_c, g_c)
                issue_write(k, rd_c, rs_c, w_c)

            @pl.when(k % 2 == 0)
            def _():
                phase(rd0, rs0, g0, w0, rd1, rs1, g1, w1)

            @pl.when(k % 2 == 1)
            def _():
                phase(rd1, rs1, g1, w1, rd0, rs0, g0, w0)

            return carry

        lax.fori_loop(0, nch, step, 0)
        wait_write(rd0 if nch % 2 == 1 else rd1,
                   rs0 if nch % 2 == 1 else rs1,
                   w0 if nch % 2 == 1 else w1)

    return pl.kernel(
        body,
        out_type=(jax.ShapeDtypeStruct((ne, DP), jnp.uint32),
                  jax.ShapeDtypeStruct((ne, DP), jnp.uint32)),
        mesh=_mesh,
        scratch_types=[
            pltpu.VMEM((epw,), jnp.int32),
            pltpu.VMEM((epw,), jnp.int32),
            pltpu.VMEM((CG, DP), jnp.uint32),
            pltpu.VMEM((CG, DP), jnp.uint32),
            pltpu.VMEM((CG, DP), jnp.uint32),
            pltpu.VMEM((CG, DP), jnp.uint32),
            pltpu.SemaphoreType.DMA,
            pltpu.SemaphoreType.DMA,
            pltpu.SemaphoreType.DMA,
            pltpu.SemaphoreType.DMA,
        ],
    )


_gather_full = _make_gather(0, EPW, E)


def _make_scatter(glob_off, ept):
    nch = ept // CS

    def body(m_lo, m_hi, dst, agg_lo, agg_hi,
             idx0, idx1, buf0, buf1, zbuf, acc, l0, l1):
        c = lax.axis_index("c")
        s = lax.axis_index("s")

        def zrow(r, carry):
            for j in range(8):
                zbuf[r, pl.ds(j * 16, 16)] = jnp.zeros((16,), jnp.float32)
            return carry

        lax.fori_loop(0, ZC, zrow, 0)
        for k in range((NZCH + NS - 1) // NS):
            cid = s + NS * k

            @pl.when(cid < NZCH)
            def _():
                pltpu.sync_copy(zbuf, acc.at[pl.ds(cid * ZC, ZC)])
        plsc.subcore_barrier()

        def run(m_ref):
            mbase = s * ept
            dbase = glob_off + s * ept

            def issue_load(k, idx_b, buf_b, sem):
                pltpu.async_copy(dst.at[pl.ds(dbase + k * CS, CS)], idx_b, sem)
                pltpu.async_copy(m_ref.at[pl.ds(mbase + k * CS, CS)], buf_b, sem)

            def wait_load(idx_b, buf_b, sem):
                pltpu.make_async_copy(dst.at[pl.ds(dbase, CS)], idx_b, sem).wait()
                pltpu.make_async_copy(m_ref.at[pl.ds(mbase, CS)], buf_b, sem).wait()

            issue_load(0, idx0, buf0, l0)

            def step(k, carry):
                def phase(idx_c, buf_c, l_c, idx_n, buf_n, l_n):
                    @pl.when(k < nch - 1)
                    def _():
                        issue_load(k + 1, idx_n, buf_n, l_n)

                    wait_load(idx_c, buf_c, l_c)
                    pltpu.sync_copy(buf_c, acc.at[idx_c], add=True)

                @pl.when(k % 2 == 0)
                def _():
                    phase(idx0, buf0, l0, idx1, buf1, l1)

                @pl.when(k % 2 == 1)
                def _():
                    phase(idx1, buf1, l1, idx0, buf0, l0)

                return carry

            lax.fori_loop(0, nch, step, 0)

        @pl.when(c == 0)
        def _():
            run(m_lo)

        @pl.when(c == 1)
        def _():
            run(m_hi)

        plsc.subcore_barrier()
        for k in range((NZCH + NS - 1) // NS):
            cid = s + NS * k

            @pl.when((cid < NZCH) & (c == 0))
            def _():
                pltpu.sync_copy(acc.at[pl.ds(cid * ZC, ZC)],
                                agg_lo.at[pl.ds(cid * ZC, ZC)])

            @pl.when((cid < NZCH) & (c == 1))
            def _():
                pltpu.sync_copy(acc.at[pl.ds(cid * ZC, ZC)],
                                agg_hi.at[pl.ds(cid * ZC, ZC)])

    return pl.kernel(
        body,
        out_type=(jax.ShapeDtypeStruct((N, 128), jnp.float32),
                  jax.ShapeDtypeStruct((N, 128), jnp.float32)),
        mesh=_mesh,
        scratch_types=[
            pltpu.VMEM((CS,), jnp.int32),
            pltpu.VMEM((CS,), jnp.int32),
            pltpu.VMEM((CS, 128), jnp.float32),
            pltpu.VMEM((CS, 128), jnp.float32),
            pltpu.VMEM((ZC, 128), jnp.float32),
            pltpu.VMEM_SHARED((N, 128), jnp.float32),
            pltpu.SemaphoreType.DMA,
            pltpu.SemaphoreType.DMA,
        ],
    )


_scatter_full = _make_scatter(0, EPT)


def _softplus(v):
    return jnp.maximum(v, 0.0) + jnp.log1p(jnp.exp(-jnp.abs(v)))


def _sigmoid(v):
    return 0.5 * jnp.tanh(0.5 * v) + 0.5


def _packpair(p):
    ug = lax.bitcast_convert_type(p[:, :DN], jnp.uint32)
    uc = lax.bitcast_convert_type(p[:, DN:], jnp.uint32)
    g16 = (ug + 0x7FFF + ((ug >> 16) & 1)) >> 16
    c16 = (uc + 0x7FFF + ((uc >> 16) & 1)) & jnp.uint32(0xFFFF0000)
    return c16 | g16


def _unpack_lo(u):
    return lax.bitcast_convert_type(u << 16, jnp.float32)


def _unpack_hi(u):
    return lax.bitcast_convert_type(u & jnp.uint32(0xFFFF0000), jnp.float32)


BN1 = 400
BE = 1000


def _embed_body(x_ref, wemb_ref, bemb_ref, wd_ref, ws_ref, h_ref, td_ref, ts_ref):
    h = jnp.dot(x_ref[...], wemb_ref[...],
                preferred_element_type=jnp.float32) + bemb_ref[...]
    h_ref[...] = h
    td_ref[...] = _packpair(jnp.dot(h, wd_ref[...],
                                    preferred_element_type=jnp.float32))
    ts_ref[...] = _packpair(jnp.dot(h, ws_ref[...],
                                    preferred_element_type=jnp.float32))


_embed = pl.pallas_call(
    _embed_body,
    grid=(N // BN1,),
    in_specs=[
        pl.BlockSpec((BN1, 128), lambda i: (i, 0)),
        pl.BlockSpec((128, DN), lambda i: (0, 0)),
        pl.BlockSpec((1, DN), lambda i: (0, 0)),
        pl.BlockSpec((DN, DZ), lambda i: (0, 0)),
        pl.BlockSpec((DN, DZ), lambda i: (0, 0)),
    ],
    out_specs=(
        pl.BlockSpec((BN1, DN), lambda i: (i, 0)),
        pl.BlockSpec((BN1, DP), lambda i: (i, 0)),
        pl.BlockSpec((BN1, DP), lambda i: (i, 0)),
    ),
    out_shape=(
        jax.ShapeDtypeStruct((N, DN), jnp.float32),
        jax.ShapeDtypeStruct((N, DP), jnp.uint32),
        jax.ShapeDtypeStruct((N, DP), jnp.uint32),
    ),
)


def _edge_body(gd_ref, gs_ref, ea_ref, wec_ref, bc_ref, mlo_ref, mhi_ref):
    efs = bc_ref[...] + jnp.dot(
        ea_ref[...], wec_ref[...], preferred_element_type=jnp.float32)
    gd = gd_ref[...]
    gs = gs_ref[...]
    gate_pre = _unpack_lo(gd) + _unpack_lo(gs) + efs[:, :DN]
    core_pre = _unpack_hi(gd) + _unpack_hi(gs) + efs[:, DN:]
    m = _sigmoid(gate_pre) * _softplus(core_pre)
    mlo_ref[...] = m[:, :128]
    mhi_ref[...] = m[:, 128:]


BE2 = 640


def _make_edge(ne):
    return pl.pallas_call(
        _edge_body,
        grid=(ne // BE2,),
        in_specs=[
            pl.BlockSpec((BE2, DP), lambda i: (i, 0)),
            pl.BlockSpec((BE2, DP), lambda i: (i, 0)),
            pl.BlockSpec((BE2, 16), lambda i: (i, 0)),
            pl.BlockSpec((16, DZ), lambda i: (0, 0)),
            pl.BlockSpec((1, DZ), lambda i: (0, 0)),
        ],
        out_specs=(
            pl.BlockSpec((BE2, 128), lambda i: (i, 0)),
            pl.BlockSpec((BE2, 128), lambda i: (i, 0)),
        ),
        out_shape=(
            jax.ShapeDtypeStruct((ne, 128), jnp.float32),
            jax.ShapeDtypeStruct((ne, 128), jnp.float32),
        ),
    )


_edge_full = _make_edge(E)


def _agg_sum(a_lo, a_hi):
    return jnp.concatenate([a_lo[...], a_hi[...]], axis=1)


def _bn_hn(st, a, h_ref, gm_ref, bt_ref):
    mu = st[0:1, :] / N
    var = st[1:2, :] / N - mu * mu
    bn = (a - mu) * lax.rsqrt(var + EPS) * gm_ref[...] + bt_ref[...]
    return bn + h_ref[...]


def _bn_stats_phase(a, st_acc):
    i = pl.program_id(1)
    blk = jnp.concatenate([
        jnp.sum(a, axis=0, keepdims=True),
        jnp.sum(a * a, axis=0, keepdims=True),
    ], axis=0)

    @pl.when(i == 0)
    def _():
        st_acc[...] = blk

    @pl.when(i > 0)
    def _():
        st_acc[...] += blk


def _bnproj_body(alo_a, ahi_a, h_ref, gm_ref, bt_ref, wd_ref,
                 ws_ref, hn_ref, td_ref, ts_ref, st_acc):
    p = pl.program_id(0)
    a = _agg_sum(alo_a, ahi_a)

    @pl.when(p == 0)
    def _():
        _bn_stats_phase(a, st_acc)

    @pl.when(p == 1)
    def _():
        hn = _bn_hn(st_acc[...], a, h_ref, gm_ref, bt_ref)
        hn_ref[...] = hn
        td_ref[...] = _packpair(jnp.dot(hn, wd_ref[...],
                                        preferred_element_type=jnp.float32))
        ts_ref[...] = _packpair(jnp.dot(hn, ws_ref[...],
                                        preferred_element_type=jnp.float32))


_bnproj = pl.pallas_call(
    _bnproj_body,
    grid=(2, N // BN1),
    in_specs=[
        pl.BlockSpec((BN1, 128), lambda p, i: (i, 0)),
        pl.BlockSpec((BN1, 128), lambda p, i: (i, 0)),
        pl.BlockSpec((BN1, DN), lambda p, i: (i, 0)),
        pl.BlockSpec((1, DN), lambda p, i: (0, 0)),
        pl.BlockSpec((1, DN), lambda p, i: (0, 0)),
        pl.BlockSpec((DN, DZ), lambda p, i: (0, 0)),
        pl.BlockSpec((DN, DZ), lambda p, i: (0, 0)),
    ],
    out_specs=(
        pl.BlockSpec((BN1, DN), lambda p, i: (i, 0)),
        pl.BlockSpec((BN1, DP), lambda p, i: (i, 0)),
        pl.BlockSpec((BN1, DP), lambda p, i: (i, 0)),
    ),
    out_shape=(
        jax.ShapeDtypeStruct((N, DN), jnp.float32),
        jax.ShapeDtypeStruct((N, DP), jnp.uint32),
        jax.ShapeDtypeStruct((N, DP), jnp.uint32),
    ),
    scratch_shapes=[pltpu.VMEM((2, DN), jnp.float32)],
)


def _bnlast_body(alo_a, ahi_a, h_ref, gm_ref, bt_ref, hn_ref,
                 st_acc):
    p = pl.program_id(0)
    a = _agg_sum(alo_a, ahi_a)

    @pl.when(p == 0)
    def _():
        _bn_stats_phase(a, st_acc)

    @pl.when(p == 1)
    def _():
        hn_ref[...] = _bn_hn(st_acc[...], a, h_ref, gm_ref, bt_ref)


_bnlast = pl.pallas_call(
    _bnlast_body,
    grid=(2, N // BN1),
    in_specs=[
        pl.BlockSpec((BN1, 128), lambda p, i: (i, 0)),
        pl.BlockSpec((BN1, 128), lambda p, i: (i, 0)),
        pl.BlockSpec((BN1, DN), lambda p, i: (i, 0)),
        pl.BlockSpec((1, DN), lambda p, i: (0, 0)),
        pl.BlockSpec((1, DN), lambda p, i: (0, 0)),
    ],
    out_specs=pl.BlockSpec((BN1, DN), lambda p, i: (i, 0)),
    out_shape=jax.ShapeDtypeStruct((N, DN), jnp.float32),
    scratch_shapes=[pltpu.VMEM((2, DN), jnp.float32)],
)


def _pool_body(b_ref, h_ref, wfc_ref, bfc_ref, wout_ref, bout_ref, o_ref,
               s_acc, c_acc):
    i = pl.program_id(0)

    @pl.when(i == 0)
    def _():
        s_acc[...] = jnp.zeros((G, DN), jnp.float32)
        c_acc[...] = jnp.zeros((G, 128), jnp.float32)

    b = b_ref[0, 0, :]
    oh = (b[None, :] == lax.broadcasted_iota(jnp.int32, (G, 1), 0)
          ).astype(jnp.float32)
    s_acc[...] += jnp.dot(oh, h_ref[...], preferred_element_type=jnp.float32)
    c_acc[...] += jnp.broadcast_to(
        jnp.sum(oh, axis=1, keepdims=True), (G, 128))

    @pl.when(i == pl.num_programs(0) - 1)
    def _():
        cnt = c_acc[:, 0:1]
        pooled = s_acc[...] / jnp.maximum(cnt, 1.0)
        cr = _softplus(pooled)
        t = _softplus(jnp.dot(cr, wfc_ref[...],
                              preferred_element_type=jnp.float32) + bfc_ref[...])
        o_ref[...] = jnp.dot(t, wout_ref[...],
                             preferred_element_type=jnp.float32) + bout_ref[...]


_pool = pl.pallas_call(
    _pool_body,
    grid=(N // BN1,),
    in_specs=[
        pl.BlockSpec((1, 1, BN1), lambda i: (i, 0, 0)),
        pl.BlockSpec((BN1, DN), lambda i: (i, 0)),
        pl.BlockSpec((DN, 128), lambda i: (0, 0)),
        pl.BlockSpec((1, 128), lambda i: (0, 0)),
        pl.BlockSpec((128, 1), lambda i: (0, 0)),
        pl.BlockSpec((1, 1), lambda i: (0, 0)),
    ],
    out_specs=pl.BlockSpec((G, 1), lambda i: (0, 0)),
    out_shape=jax.ShapeDtypeStruct((G, 1), jnp.float32),
    scratch_shapes=[
        pltpu.VMEM((G, DN), jnp.float32),
        pltpu.VMEM((G, 128), jnp.float32),
    ],
)


def kernel(x, edge_index, edge_attr, batch, W_emb, b_emb, Wf, bf, Ws, bs,
           gamma, beta, W_fc, b_fc, W_out, b_out):
    src = edge_index[0].astype(jnp.int32)
    dst = edge_index[1].astype(jnp.int32)
    batch3 = batch.astype(jnp.int32).reshape(N // BN1, 1, BN1)

    L = Wf.shape[0]
    Wd = [jnp.concatenate([Wf[l, :DN], Ws[l, :DN]], axis=1) for l in range(L)]
    Wsc = [jnp.concatenate([Wf[l, DN:2 * DN], Ws[l, DN:2 * DN]], axis=1)
           for l in range(L)]
    Wec = [jnp.concatenate([Wf[l, 2 * DN:], Ws[l, 2 * DN:]], axis=1)
           for l in range(L)]
    bc = [jnp.concatenate([bf[l], bs[l]]).reshape(1, DZ) for l in range(L)]

    h, td, ts = _embed(x, W_emb, b_emb.reshape(1, DN), Wd[0], Wsc[0])
    for l in range(L):
        gd, gs = _gather_full(td, ts, dst, src)
        mlo, mhi = _edge_full(gd, gs, edge_attr, Wec[l], bc[l])
        alo, ahi = _scatter_full(mlo, mhi, dst)
        gm = gamma[l].reshape(1, DN)
        bt = beta[l].reshape(1, DN)
        if l + 1 < L:
            h, td, ts = _bnproj(alo, ahi, h, gm, bt,
                                Wd[l + 1], Wsc[l + 1])
        else:
            h = _bnlast(alo, ahi, h, gm, bt)
    return _pool(batch3, h, W_fc, b_fc.reshape(1, 128), W_out,
                 b_out.reshape(1, 1))


# speedup vs baseline: 1.2153x; 1.2153x over previous
import functools

import jax
import jax.numpy as jnp
from jax import lax
from jax.experimental import pallas as pl
from jax.experimental.pallas import tpu as pltpu
from jax.experimental.pallas import tpu_sc as plsc

N = 10000
E = 160000
DN = 256
DZ = 512
G = 64
EPS = 1e-5

DP = 256
NC, NS = 2, 16
NW = NC * NS
EPW = E // NW
CG = 40
EPT = E // NS
CS = 80
ZC = 80
NZCH = N // ZC

_mesh = plsc.VectorSubcoreMesh(
    core_axis_name="c", subcore_axis_name="s", num_cores=NC, num_subcores=NS)


EH_A = 81920
EH_B = E - EH_A
EPW_A = EH_A // NW
EPW_B = EH_B // NW
EPT_A = EH_A // NS
EPT_B = EH_B // NS


def _make_gather(glob_off, epw, ne):
    nch = epw // CG

    def body(td, ts, dst, src, gd, gs, idx_d, idx_s,
             rd0, rs0, rd1, rs1, g0, g1, w0, w1):
        c = lax.axis_index("c")
        s = lax.axis_index("s")
        wid = s * NC + c
        base = glob_off + wid * epw
        obase = wid * epw
        pltpu.sync_copy(dst.at[pl.ds(base, epw)], idx_d)
        pltpu.sync_copy(src.at[pl.ds(base, epw)], idx_s)

        def issue_gather(k, rd, rs, sem):
            pltpu.async_copy(td.at[idx_d.at[pl.ds(k * CG, CG)]], rd, sem)
            pltpu.async_copy(ts.at[idx_s.at[pl.ds(k * CG, CG)]], rs, sem)

        def wait_gather(rd, rs, sem):
            pltpu.make_async_copy(td.at[idx_d.at[pl.ds(0, CG)]], rd, sem).wait()
            pltpu.make_async_copy(ts.at[idx_s.at[pl.ds(0, CG)]], rs, sem).wait()

        def issue_write(k, rd, rs, sem):
            off = obase + k * CG
            pltpu.async_copy(rd, gd.at[pl.ds(off, CG)], sem)
            pltpu.async_copy(rs, gs.at[pl.ds(off, CG)], sem)

        def wait_write(rd, rs, sem):
            pltpu.make_async_copy(rd, gd.at[pl.ds(obase, CG)], sem).wait()
            pltpu.make_async_copy(rs, gs.at[pl.ds(obase, CG)], sem).wait()

        issue_gather(0, rd0, rs0, g0)

        def step(k, carry):
            def phase(rd_c, rs_c, g_c, w_c, rd_n, rs_n, g_n, w_n):
                @pl.when(k > 0)
                def _():
                    wait_write(rd_n, rs_n, w_n)

                @pl.when(k < nch - 1)
                def _():
                    issue_gather(k + 1, rd_n, rs_n, g_n)

                wait_gather(rd_c, rs_c, g_c)
                issue_write(k, rd_c, rs_c, w_c)

            @pl.when(k % 2 == 0)
            def _():
                phase(rd0, rs0, g0, w0, rd1, rs1, g1, w1)

            @pl.when(k % 2 == 1)
            def _():
                phase(rd1, rs1, g1, w1, rd0, rs0, g0, w0)

            return carry

        lax.fori_loop(0, nch, step, 0)
        wait_write(rd0 if nch % 2 == 1 else rd1,
                   rs0 if nch % 2 == 1 else rs1,
                   w0 if nch % 2 == 1 else w1)

    return pl.kernel(
        body,
        out_type=(jax.ShapeDtypeStruct((ne, DP), jnp.uint32),
                  jax.ShapeDtypeStruct((ne, DP), jnp.uint32)),
        mesh=_mesh,
        scratch_types=[
            pltpu.VMEM((epw,), jnp.int32),
            pltpu.VMEM((epw,), jnp.int32),
            pltpu.VMEM((CG, DP), jnp.uint32),
            pltpu.VMEM((CG, DP), jnp.uint32),
            pltpu.VMEM((CG, DP), jnp.uint32),
            pltpu.VMEM((CG, DP), jnp.uint32),
            pltpu.SemaphoreType.DMA,
            pltpu.SemaphoreType.DMA,
            pltpu.SemaphoreType.DMA,
            pltpu.SemaphoreType.DMA,
        ],
    )


_gather_a = _make_gather(0, EPW_A, EH_A)
_gather_b = _make_gather(EH_A, EPW_B, EH_B)


def _make_scatter(glob_off, ept):
    nch = ept // CS

    def body(m_lo, m_hi, dst, agg_lo, agg_hi,
             idx0, idx1, buf0, buf1, zbuf, acc, l0, l1):
        c = lax.axis_index("c")
        s = lax.axis_index("s")

        def zrow(r, carry):
            for j in range(8):
                zbuf[r, pl.ds(j * 16, 16)] = jnp.zeros((16,), jnp.float32)
            return carry

        lax.fori_loop(0, ZC, zrow, 0)
        for k in range((NZCH + NS - 1) // NS):
            cid = s + NS * k

            @pl.when(cid < NZCH)
            def _():
                pltpu.sync_copy(zbuf, acc.at[pl.ds(cid * ZC, ZC)])
        plsc.subcore_barrier()

        def run(m_ref):
            mbase = s * ept
            dbase = glob_off + s * ept

            def issue_load(k, idx_b, buf_b, sem):
                pltpu.async_copy(dst.at[pl.ds(dbase + k * CS, CS)], idx_b, sem)
                pltpu.async_copy(m_ref.at[pl.ds(mbase + k * CS, CS)], buf_b, sem)

            def wait_load(idx_b, buf_b, sem):
                pltpu.make_async_copy(dst.at[pl.ds(dbase, CS)], idx_b, sem).wait()
                pltpu.make_async_copy(m_ref.at[pl.ds(mbase, CS)], buf_b, sem).wait()

            issue_load(0, idx0, buf0, l0)

            def step(k, carry):
                def phase(idx_c, buf_c, l_c, idx_n, buf_n, l_n):
                    @pl.when(k < nch - 1)
                    def _():
                        issue_load(k + 1, idx_n, buf_n, l_n)

                    wait_load(idx_c, buf_c, l_c)
                    pltpu.sync_copy(buf_c, acc.at[idx_c], add=True)

                @pl.when(k % 2 == 0)
                def _():
                    phase(idx0, buf0, l0, idx1, buf1, l1)

                @pl.when(k % 2 == 1)
                def _():
                    phase(idx1, buf1, l1, idx0, buf0, l0)

                return carry

            lax.fori_loop(0, nch, step, 0)

        @pl.when(c == 0)
        def _():
            run(m_lo)

        @pl.when(c == 1)
        def _():
            run(m_hi)

        plsc.subcore_barrier()
        for k in range((NZCH + NS - 1) // NS):
            cid = s + NS * k

            @pl.when((cid < NZCH) & (c == 0))
            def _():
                pltpu.sync_copy(acc.at[pl.ds(cid * ZC, ZC)],
                                agg_lo.at[pl.ds(cid * ZC, ZC)])

            @pl.when((cid < NZCH) & (c == 1))
            def _():
                pltpu.sync_copy(acc.at[pl.ds(cid * ZC, ZC)],
                                agg_hi.at[pl.ds(cid * ZC, ZC)])

    return pl.kernel(
        body,
        out_type=(jax.ShapeDtypeStruct((N, 128), jnp.float32),
                  jax.ShapeDtypeStruct((N, 128), jnp.float32)),
        mesh=_mesh,
        scratch_types=[
            pltpu.VMEM((CS,), jnp.int32),
            pltpu.VMEM((CS,), jnp.int32),
            pltpu.VMEM((CS, 128), jnp.float32),
            pltpu.VMEM((CS, 128), jnp.float32),
            pltpu.VMEM((ZC, 128), jnp.float32),
            pltpu.VMEM_SHARED((N, 128), jnp.float32),
            pltpu.SemaphoreType.DMA,
            pltpu.SemaphoreType.DMA,
        ],
    )


_scatter_a = _make_scatter(0, EPT_A)
_scatter_b = _make_scatter(EH_A, EPT_B)


def _softplus(v):
    return jnp.maximum(v, 0.0) + jnp.log1p(jnp.exp(-jnp.abs(v)))


def _sigmoid(v):
    return 0.5 * jnp.tanh(0.5 * v) + 0.5


def _packpair(p):
    ug = lax.bitcast_convert_type(p[:, :DN], jnp.uint32)
    uc = lax.bitcast_convert_type(p[:, DN:], jnp.uint32)
    g16 = (ug + 0x7FFF + ((ug >> 16) & 1)) >> 16
    c16 = (uc + 0x7FFF + ((uc >> 16) & 1)) & jnp.uint32(0xFFFF0000)
    return c16 | g16


def _unpack_lo(u):
    return lax.bitcast_convert_type(u << 16, jnp.float32)


def _unpack_hi(u):
    return lax.bitcast_convert_type(u & jnp.uint32(0xFFFF0000), jnp.float32)


BN1 = 400
BE = 1000


def _embed_body(x_ref, wemb_ref, bemb_ref, wd_ref, ws_ref, h_ref, td_ref, ts_ref):
    h = jnp.dot(x_ref[...], wemb_ref[...],
                preferred_element_type=jnp.float32) + bemb_ref[...]
    h_ref[...] = h
    hb = h.astype(jnp.bfloat16)
    td_ref[...] = _packpair(jnp.dot(hb, wd_ref[...],
                                    preferred_element_type=jnp.float32))
    ts_ref[...] = _packpair(jnp.dot(hb, ws_ref[...],
                                    preferred_element_type=jnp.float32))


_embed = pl.pallas_call(
    _embed_body,
    grid=(N // BN1,),
    in_specs=[
        pl.BlockSpec((BN1, 128), lambda i: (i, 0)),
        pl.BlockSpec((128, DN), lambda i: (0, 0)),
        pl.BlockSpec((1, DN), lambda i: (0, 0)),
        pl.BlockSpec((DN, DZ), lambda i: (0, 0)),
        pl.BlockSpec((DN, DZ), lambda i: (0, 0)),
    ],
    out_specs=(
        pl.BlockSpec((BN1, DN), lambda i: (i, 0)),
        pl.BlockSpec((BN1, DP), lambda i: (i, 0)),
        pl.BlockSpec((BN1, DP), lambda i: (i, 0)),
    ),
    out_shape=(
        jax.ShapeDtypeStruct((N, DN), jnp.float32),
        jax.ShapeDtypeStruct((N, DP), jnp.uint32),
        jax.ShapeDtypeStruct((N, DP), jnp.uint32),
    ),
)


def _edge_body(gd_ref, gs_ref, ea_ref, wec_ref, bc_ref, mlo_ref, mhi_ref):
    efs = bc_ref[...] + jnp.dot(
        ea_ref[...], wec_ref[...], preferred_element_type=jnp.float32)
    gd = gd_ref[...]
    gs = gs_ref[...]
    gate_pre = _unpack_lo(gd) + _unpack_lo(gs) + efs[:, :DN]
    core_pre = _unpack_hi(gd) + _unpack_hi(gs) + efs[:, DN:]
    m = _sigmoid(gate_pre) * _softplus(core_pre)
    mlo_ref[...] = m[:, :128]
    mhi_ref[...] = m[:, 128:]


BE2 = 1280


def _make_edge(ne):
    return pl.pallas_call(
        _edge_body,
        grid=(ne // BE2,),
        in_specs=[
            pl.BlockSpec((BE2, DP), lambda i: (i, 0)),
            pl.BlockSpec((BE2, DP), lambda i: (i, 0)),
            pl.BlockSpec((BE2, 16), lambda i: (i, 0)),
            pl.BlockSpec((16, DZ), lambda i: (0, 0)),
            pl.BlockSpec((1, DZ), lambda i: (0, 0)),
        ],
        out_specs=(
            pl.BlockSpec((BE2, 128), lambda i: (i, 0)),
            pl.BlockSpec((BE2, 128), lambda i: (i, 0)),
        ),
        out_shape=(
            jax.ShapeDtypeStruct((ne, 128), jnp.float32),
            jax.ShapeDtypeStruct((ne, 128), jnp.float32),
        ),
    )


_edge_a = _make_edge(EH_A)
_edge_b = _make_edge(EH_B)


def _agg_sum(a_lo, a_hi, b_lo, b_hi):
    return jnp.concatenate([a_lo[...] + b_lo[...], a_hi[...] + b_hi[...]],
                           axis=1)


def _bn_hn(st, a, h_ref, gm_ref, bt_ref):
    mu = st[0:1, :] / N
    var = st[1:2, :] / N - mu * mu
    bn = (a - mu) * lax.rsqrt(var + EPS) * gm_ref[...] + bt_ref[...]
    return bn + h_ref[...]


def _bn_stats_phase(a, st_acc):
    i = pl.program_id(1)
    blk = jnp.concatenate([
        jnp.sum(a, axis=0, keepdims=True),
        jnp.sum(a * a, axis=0, keepdims=True),
    ], axis=0)

    @pl.when(i == 0)
    def _():
        st_acc[...] = blk

    @pl.when(i > 0)
    def _():
        st_acc[...] += blk


def _bnproj_body(alo_a, ahi_a, alo_b, ahi_b, h_ref, gm_ref, bt_ref, wd_ref,
                 ws_ref, hn_ref, td_ref, ts_ref, st_acc):
    p = pl.program_id(0)
    a = _agg_sum(alo_a, ahi_a, alo_b, ahi_b)

    @pl.when(p == 0)
    def _():
        _bn_stats_phase(a, st_acc)

    @pl.when(p == 1)
    def _():
        hn = _bn_hn(st_acc[...], a, h_ref, gm_ref, bt_ref)
        hn_ref[...] = hn
        hb = hn.astype(jnp.bfloat16)
        td_ref[...] = _packpair(jnp.dot(hb, wd_ref[...],
                                        preferred_element_type=jnp.float32))
        ts_ref[...] = _packpair(jnp.dot(hb, ws_ref[...],
                                        preferred_element_type=jnp.float32))


_bnproj = pl.pallas_call(
    _bnproj_body,
    grid=(2, N // BN1),
    in_specs=[
        pl.BlockSpec((BN1, 128), lambda p, i: (i, 0)),
        pl.BlockSpec((BN1, 128), lambda p, i: (i, 0)),
        pl.BlockSpec((BN1, 128), lambda p, i: (i, 0)),
        pl.BlockSpec((BN1, 128), lambda p, i: (i, 0)),
        pl.BlockSpec((BN1, DN), lambda p, i: (i, 0)),
        pl.BlockSpec((1, DN), lambda p, i: (0, 0)),
        pl.BlockSpec((1, DN), lambda p, i: (0, 0)),
        pl.BlockSpec((DN, DZ), lambda p, i: (0, 0)),
        pl.BlockSpec((DN, DZ), lambda p, i: (0, 0)),
    ],
    out_specs=(
        pl.BlockSpec((BN1, DN), lambda p, i: (i, 0)),
        pl.BlockSpec((BN1, DP), lambda p, i: (i, 0)),
        pl.BlockSpec((BN1, DP), lambda p, i: (i, 0)),
    ),
    out_shape=(
        jax.ShapeDtypeStruct((N, DN), jnp.float32),
        jax.ShapeDtypeStruct((N, DP), jnp.uint32),
        jax.ShapeDtypeStruct((N, DP), jnp.uint32),
    ),
    scratch_shapes=[pltpu.VMEM((2, DN), jnp.float32)],
)


def _bnlast_body(alo_a, ahi_a, alo_b, ahi_b, h_ref, gm_ref, bt_ref, hn_ref,
                 st_acc):
    p = pl.program_id(0)
    a = _agg_sum(alo_a, ahi_a, alo_b, ahi_b)

    @pl.when(p == 0)
    def _():
        _bn_stats_phase(a, st_acc)

    @pl.when(p == 1)
    def _():
        hn_ref[...] = _bn_hn(st_acc[...], a, h_ref, gm_ref, bt_ref)


_bnlast = pl.pallas_call(
    _bnlast_body,
    grid=(2, N // BN1),
    in_specs=[
        pl.BlockSpec((BN1, 128), lambda p, i: (i, 0)),
        pl.BlockSpec((BN1, 128), lambda p, i: (i, 0)),
        pl.BlockSpec((BN1, 128), lambda p, i: (i, 0)),
        pl.BlockSpec((BN1, 128), lambda p, i: (i, 0)),
        pl.BlockSpec((BN1, DN), lambda p, i: (i, 0)),
        pl.BlockSpec((1, DN), lambda p, i: (0, 0)),
        pl.BlockSpec((1, DN), lambda p, i: (0, 0)),
    ],
    out_specs=pl.BlockSpec((BN1, DN), lambda p, i: (i, 0)),
    out_shape=jax.ShapeDtypeStruct((N, DN), jnp.float32),
    scratch_shapes=[pltpu.VMEM((2, DN), jnp.float32)],
)


def _pool_body(b_ref, h_ref, wfc_ref, bfc_ref, wout_ref, bout_ref, o_ref,
               s_acc, c_acc):
    i = pl.program_id(0)

    @pl.when(i == 0)
    def _():
        s_acc[...] = jnp.zeros((G, DN), jnp.float32)
        c_acc[...] = jnp.zeros((G, 128), jnp.float32)

    b = b_ref[0, 0, :]
    oh = (b[None, :] == lax.broadcasted_iota(jnp.int32, (G, 1), 0)
          ).astype(jnp.float32)
    s_acc[...] += jnp.dot(oh, h_ref[...], preferred_element_type=jnp.float32)
    c_acc[...] += jnp.broadcast_to(
        jnp.sum(oh, axis=1, keepdims=True), (G, 128))

    @pl.when(i == pl.num_programs(0) - 1)
    def _():
        cnt = c_acc[:, 0:1]
        pooled = s_acc[...] / jnp.maximum(cnt, 1.0)
        cr = _softplus(pooled)
        t = _softplus(jnp.dot(cr, wfc_ref[...],
                              preferred_element_type=jnp.float32) + bfc_ref[...])
        o_ref[...] = jnp.dot(t, wout_ref[...],
                             preferred_element_type=jnp.float32) + bout_ref[...]


_pool = pl.pallas_call(
    _pool_body,
    grid=(N // BN1,),
    in_specs=[
        pl.BlockSpec((1, 1, BN1), lambda i: (i, 0, 0)),
        pl.BlockSpec((BN1, DN), lambda i: (i, 0)),
        pl.BlockSpec((DN, 128), lambda i: (0, 0)),
        pl.BlockSpec((1, 128), lambda i: (0, 0)),
        pl.BlockSpec((128, 1), lambda i: (0, 0)),
        pl.BlockSpec((1, 1), lambda i: (0, 0)),
    ],
    out_specs=pl.BlockSpec((G, 1), lambda i: (0, 0)),
    out_shape=jax.ShapeDtypeStruct((G, 1), jnp.float32),
    scratch_shapes=[
        pltpu.VMEM((G, DN), jnp.float32),
        pltpu.VMEM((G, 128), jnp.float32),
    ],
)


def kernel(x, edge_index, edge_attr, batch, W_emb, b_emb, Wf, bf, Ws, bs,
           gamma, beta, W_fc, b_fc, W_out, b_out):
    src = edge_index[0].astype(jnp.int32)
    dst = edge_index[1].astype(jnp.int32)
    batch3 = batch.astype(jnp.int32).reshape(N // BN1, 1, BN1)

    L = Wf.shape[0]
    Wd = [jnp.concatenate([Wf[l, :DN], Ws[l, :DN]],
                          axis=1).astype(jnp.bfloat16) for l in range(L)]
    Wsc = [jnp.concatenate([Wf[l, DN:2 * DN], Ws[l, DN:2 * DN]],
                           axis=1).astype(jnp.bfloat16) for l in range(L)]
    Wec = [jnp.concatenate([Wf[l, 2 * DN:], Ws[l, 2 * DN:]], axis=1)
           for l in range(L)]
    bc = [jnp.concatenate([bf[l], bs[l]]).reshape(1, DZ) for l in range(L)]

    ea_a = edge_attr[:EH_A]
    ea_b = edge_attr[EH_A:]
    h, td, ts = _embed(x, W_emb, b_emb.reshape(1, DN), Wd[0], Wsc[0])
    for l in range(L):
        gd_a, gs_a = _gather_a(td, ts, dst, src)
        gd_b, gs_b = _gather_b(td, ts, dst, src)
        mlo_a, mhi_a = _edge_a(gd_a, gs_a, ea_a, Wec[l], bc[l])
        mlo_b, mhi_b = _edge_b(gd_b, gs_b, ea_b, Wec[l], bc[l])
        alo_a, ahi_a = _scatter_a(mlo_a, mhi_a, dst)
        alo_b, ahi_b = _scatter_b(mlo_b, mhi_b, dst)
        gm = gamma[l].reshape(1, DN)
        bt = beta[l].reshape(1, DN)
        if l + 1 < L:
            h, td, ts = _bnproj(alo_a, ahi_a, alo_b, ahi_b, h, gm, bt,
                                Wd[l + 1], Wsc[l + 1])
        else:
            h = _bnlast(alo_a, ahi_a, alo_b, ahi_b, h, gm, bt)
    return _pool(batch3, h, W_fc, b_fc.reshape(1, 128), W_out,
                 b_out.reshape(1, 1))
